# Initial kernel scaffold; baseline (speedup 1.0000x reference)
import jax
import jax.numpy as jnp
from jax.experimental import pallas as pl


def kernel(x, table):
    raise NotImplementedError("write your pallas kernel here")


# speedup vs baseline: 1.6290x; 1.6290x over previous
import functools
import math

import jax
import jax.numpy as jnp
from jax import lax
from jax.experimental import pallas as pl
from jax.experimental.pallas import tpu as pltpu
from jax.experimental.pallas import tpu_sc as plsc

D_MODEL = 128
SCALE = math.sqrt(D_MODEL)
LANES = 16
NUM_CORES = 2
NUM_SUBCORES = 16
NUM_WORKERS = NUM_CORES * NUM_SUBCORES

B_TOTAL = 4096 * 50
IDX_LANES = 128
IDX_ROWS = B_TOTAL // IDX_LANES
K_PER_W = IDX_ROWS // NUM_WORKERS

_mesh = plsc.VectorSubcoreMesh(core_axis_name="c", subcore_axis_name="s")


@functools.partial(
    pl.kernel,
    mesh=_mesh,
    out_type=jax.ShapeDtypeStruct((B_TOTAL, D_MODEL), jnp.float32),
    scratch_types=[
        pltpu.VMEM((K_PER_W, IDX_LANES), jnp.int32),
        pltpu.VMEM((IDX_LANES, D_MODEL), jnp.float32),
        pltpu.SemaphoreType.DMA,
    ],
)
def _emb_lookup(x_hbm, table_hbm, out_hbm, idx_v, rows_v, sem):
    wid = lax.axis_index("s") * NUM_CORES + lax.axis_index("c")
    row0 = wid * K_PER_W
    pltpu.sync_copy(x_hbm.at[wid], idx_v)

    def stream_body(j, carry):
        pltpu.async_copy(table_hbm.at[idx_v.at[j]], rows_v, sem).wait()

        def group_body(g, c2):
            idx16 = idx_v[j, pl.ds(g * LANES, LANES)]
            for l in range(LANES):
                r = g * LANES + l
                s = jnp.where(idx16[l] == 0, 0.0, SCALE)
                scale = jnp.full((LANES,), s, jnp.float32)
                for c in range(D_MODEL // LANES):
                    sl = pl.ds(c * LANES, LANES)
                    rows_v[r, sl] = rows_v[r, sl] * scale
            return c2

        lax.fori_loop(0, IDX_LANES // LANES, group_body, 0)
        pltpu.sync_copy(
            rows_v, out_hbm.at[pl.ds((row0 + j) * IDX_LANES, IDX_LANES)]
        )
        return carry

    lax.fori_loop(0, K_PER_W, stream_body, 0)


def kernel(x, table):
    x_flat = x.reshape(NUM_WORKERS, K_PER_W, IDX_LANES)
    out = _emb_lookup(x_flat, table)
    return out.reshape(4096, 50, D_MODEL)


# speedup vs baseline: 3.0345x; 1.8628x over previous
import functools
import math

import jax
import jax.numpy as jnp
from jax import lax
from jax.experimental import pallas as pl
from jax.experimental.pallas import tpu as pltpu
from jax.experimental.pallas import tpu_sc as plsc

D_MODEL = 128
SCALE = math.sqrt(D_MODEL)
LANES = 16
NUM_CORES = 2
NUM_SUBCORES = 16
NUM_WORKERS = NUM_CORES * NUM_SUBCORES

BATCH = 4096
SEQ = 50
ROWS_PER_W = BATCH // NUM_WORKERS

_GROUPS = (0, 16, 32, SEQ - LANES)

_mesh = plsc.VectorSubcoreMesh(core_axis_name="c", subcore_axis_name="s")


@functools.partial(
    pl.kernel,
    mesh=_mesh,
    out_type=jax.ShapeDtypeStruct((BATCH, SEQ, D_MODEL), jnp.float32),
    scratch_types=[
        pltpu.VMEM((ROWS_PER_W, SEQ), jnp.int32),
        pltpu.VMEM((2, SEQ, D_MODEL), jnp.float32),
        pltpu.VMEM((2, SEQ, D_MODEL), jnp.float32),
        pltpu.SemaphoreType.DMA,
        pltpu.SemaphoreType.DMA,
        pltpu.SemaphoreType.DMA,
        pltpu.SemaphoreType.DMA,
    ],
)
def _emb_lookup(x_hbm, table_hbm, out_hbm, idx_v, raw_v, sc_v, g0, g1, s0, s1):
    wid = lax.axis_index("s") * NUM_CORES + lax.axis_index("c")
    b0 = wid * ROWS_PER_W
    gsem = (g0, g1)
    ssem = (s0, s1)

    pltpu.sync_copy(x_hbm.at[pl.ds(b0, ROWS_PER_W)], idx_v)

    def start_gather(jj, t):
        pltpu.async_copy(table_hbm.at[idx_v.at[jj]], raw_v.at[t], gsem[t])

    def wait_gather(jj, t):
        pltpu.make_async_copy(
            table_hbm.at[idx_v.at[jj]], raw_v.at[t], gsem[t]
        ).wait()

    def wait_scatter(t):
        pltpu.make_async_copy(sc_v.at[t], out_hbm.at[b0], ssem[t]).wait()

    def compute(jj, raw, scl):
        for r0 in _GROUPS:
            idx16 = idx_v[jj, pl.ds(r0, LANES)]
            for l in range(LANES):
                r = r0 + l
                s = jnp.where(idx16[l] == 0, 0.0, SCALE)
                scale = jnp.full((LANES,), s, jnp.float32)
                for c in range(D_MODEL // LANES):
                    sl = pl.ds(c * LANES, LANES)
                    scl[r, sl] = raw[r, sl] * scale

    start_gather(0, 0)
    start_gather(1, 1)

    def body(j, carry):
        for t in range(2):
            jj = j + t
            wait_gather(jj, t)

            @pl.when(jj >= 2)
            def _():
                wait_scatter(t)

            compute(jj, raw_v.at[t], sc_v.at[t])
            pltpu.async_copy(sc_v.at[t], out_hbm.at[b0 + jj], ssem[t])

            @pl.when(jj + 2 < ROWS_PER_W)
            def _():
                start_gather(jj + 2, t)

        return carry

    lax.fori_loop(0, ROWS_PER_W // 2, lambda i, c: body(i * 2, c), 0)
    wait_scatter(0)
    wait_scatter(1)


def kernel(x, table):
    return _emb_lookup(x, table)


# speedup vs baseline: 5.4793x; 1.8057x over previous
import functools
import math

import jax
import jax.numpy as jnp
from jax import lax
from jax.experimental import pallas as pl
from jax.experimental.pallas import tpu as pltpu
from jax.experimental.pallas import tpu_sc as plsc

D_MODEL = 128
SCALE = math.sqrt(D_MODEL)
LANES = 16
NUM_CORES = 2
NUM_SUBCORES = 16
NUM_WORKERS = NUM_CORES * NUM_SUBCORES

BATCH = 4096
SEQ = 50
COLS_PER_W = BATCH // NUM_WORKERS

_mesh = plsc.VectorSubcoreMesh(core_axis_name="c", subcore_axis_name="s")


@functools.partial(
    pl.kernel,
    mesh=_mesh,
    out_type=jax.ShapeDtypeStruct((SEQ, BATCH, D_MODEL), jnp.float32),
    scratch_types=[
        pltpu.VMEM((SEQ, COLS_PER_W), jnp.int32),
        pltpu.VMEM((2, COLS_PER_W, D_MODEL), jnp.float32),
        pltpu.VMEM((2, COLS_PER_W, D_MODEL), jnp.float32),
        pltpu.SemaphoreType.DMA,
        pltpu.SemaphoreType.DMA,
        pltpu.SemaphoreType.DMA,
        pltpu.SemaphoreType.DMA,
    ],
)
def _emb_lookup(xt_hbm, table_hbm, out_hbm, idx_v, raw_v, sc_v, g0, g1, s0, s1):
    wid = lax.axis_index("s") * NUM_CORES + lax.axis_index("c")
    b0 = wid * COLS_PER_W
    gsem = (g0, g1)
    ssem = (s0, s1)

    pltpu.sync_copy(xt_hbm.at[:, pl.ds(b0, COLS_PER_W)], idx_v)

    def start_gather(ss, t):
        pltpu.async_copy(table_hbm.at[idx_v.at[ss]], raw_v.at[t], gsem[t])

    def wait_gather(ss, t):
        pltpu.make_async_copy(
            table_hbm.at[idx_v.at[ss]], raw_v.at[t], gsem[t]
        ).wait()

    def wait_scatter(t):
        pltpu.make_async_copy(
            sc_v.at[t], out_hbm.at[0, pl.ds(b0, COLS_PER_W)], ssem[t]
        ).wait()

    def compute(ss, raw, scl):
        def group(g, carry):
            r0 = g * LANES
            idx16 = idx_v[ss, pl.ds(r0, LANES)]
            for l in range(LANES):
                s = jnp.where(idx16[l] == 0, 0.0, SCALE)
                scale = jnp.full((LANES,), s, jnp.float32)
                for c in range(D_MODEL // LANES):
                    sl = pl.ds(c * LANES, LANES)
                    scl[r0 + l, sl] = raw[r0 + l, sl] * scale
            return carry

        lax.fori_loop(0, COLS_PER_W // LANES, group, 0)

    start_gather(0, 0)
    start_gather(1, 1)

    def body(j, carry):
        for t in range(2):
            ss = j + t
            wait_gather(ss, t)

            @pl.when(ss >= 2)
            def _():
                wait_scatter(t)

            compute(ss, raw_v.at[t], sc_v.at[t])
            pltpu.async_copy(
                sc_v.at[t], out_hbm.at[ss, pl.ds(b0, COLS_PER_W)], ssem[t]
            )

            @pl.when(ss + 2 < SEQ)
            def _():
                start_gather(ss + 2, t)

        return carry

    lax.fori_loop(0, SEQ // 2, lambda i, c: body(i * 2, c), 0)
    wait_scatter(0)
    wait_scatter(1)


def kernel(x, table):
    out_t = _emb_lookup(x.T, table)
    return jnp.transpose(out_t, (1, 0, 2))


# speedup vs baseline: 6.1005x; 1.1134x over previous
import functools
import math

import jax
import jax.numpy as jnp
from jax import lax
from jax.experimental import pallas as pl
from jax.experimental.pallas import tpu as pltpu
from jax.experimental.pallas import tpu_sc as plsc

D_MODEL = 128
SCALE = math.sqrt(D_MODEL)
LANES = 16
NUM_CORES = 2
NUM_SUBCORES = 16
NUM_WORKERS = NUM_CORES * NUM_SUBCORES

BATCH = 4096
SEQ = 50
COLS_PER_W = BATCH // NUM_WORKERS
NSLOT = 4

_mesh = plsc.VectorSubcoreMesh(core_axis_name="c", subcore_axis_name="s")


@functools.partial(
    pl.kernel,
    mesh=_mesh,
    out_type=jax.ShapeDtypeStruct((SEQ, BATCH, D_MODEL), jnp.float32),
    scratch_types=[
        pltpu.VMEM((SEQ, COLS_PER_W), jnp.int32),
        pltpu.VMEM((NSLOT, COLS_PER_W, D_MODEL), jnp.float32),
        pltpu.SemaphoreType.DMA,
        pltpu.SemaphoreType.DMA,
    ],
)
def _emb_lookup(xt_hbm, table_hbm, out_hbm, idx_v, ring_v, gsem, ssem):
    wid = lax.axis_index("s") * NUM_CORES + lax.axis_index("c")
    b0 = wid * COLS_PER_W

    pltpu.sync_copy(xt_hbm.at[:, pl.ds(b0, COLS_PER_W)], idx_v)

    def start_gather(ss, t):
        pltpu.async_copy(table_hbm.at[idx_v.at[ss]], ring_v.at[t], gsem)

    def wait_gather(t):
        pltpu.make_async_copy(table_hbm.at[idx_v.at[0]], ring_v.at[t], gsem).wait()

    def start_scatter(ss, t):
        pltpu.async_copy(ring_v.at[t], out_hbm.at[ss, pl.ds(b0, COLS_PER_W)], ssem)

    def wait_scatter(t):
        pltpu.make_async_copy(
            ring_v.at[t], out_hbm.at[0, pl.ds(b0, COLS_PER_W)], ssem
        ).wait()

    def compute(ss, slab):
        def group(g, carry):
            r0 = g * LANES
            idx16 = idx_v[ss, pl.ds(r0, LANES)]
            scale16 = jnp.where(idx16 == 0, 0.0, SCALE).astype(jnp.float32)
            for l in range(LANES):
                scale = jnp.full((LANES,), scale16[l], jnp.float32)
                for c in range(D_MODEL // LANES):
                    sl = pl.ds(c * LANES, LANES)
                    slab[r0 + l, sl] = slab[r0 + l, sl] * scale
            return carry

        lax.fori_loop(0, COLS_PER_W // LANES, group, 0)

    def step(ss, t):
        wait_gather(t)
        compute(ss, ring_v.at[t])
        start_scatter(ss, t)

        @pl.when(ss >= 2)
        def _():
            wait_scatter((t + 2) % NSLOT)

        @pl.when(ss + 2 < SEQ)
        def _():
            start_gather(ss + 2, (t + 2) % NSLOT)

    start_gather(0, 0)
    start_gather(1, 1)

    def body(j, carry):
        for t in range(NSLOT):
            step(j * NSLOT + t, t)
        return carry

    lax.fori_loop(0, SEQ // NSLOT, body, 0)
    step(48, 0)
    step(49, 1)
    wait_scatter(0)
    wait_scatter(1)


def kernel(x, table):
    out_t = _emb_lookup(x.T, table)
    return jnp.transpose(out_t, (1, 0, 2))
